# baseline (speedup 1.0000x reference)
import jax
import jax.numpy as jnp
from jax import lax
from jax.experimental import pallas as pl
from jax.experimental.pallas import tpu as pltpu

_B = 128
_V = 100000
_OUT = 2 * _V - 1

_G = 16
_RB_IN = 10000
_NB_IN = _V // _RB_IN
_NG = _RB_IN // _G
_RB_OUT = 25000
_NB_OUT = (_OUT + _RB_OUT - 1) // _RB_OUT

_BIG = 2**30


def _fused_body(x0_ref, x1_ref, out_ref, m0, i0, m1, i1, res_scr):
    i = pl.program_id(0)

    @pl.when(i == 0)
    def _init():
        m0[...] = jnp.full_like(m0, -1.0)
        i0[...] = jnp.zeros_like(i0)
        m1[...] = jnp.full_like(m1, -1.0)
        i1[...] = jnp.zeros_like(i1)

    @pl.when(i < _NB_IN)
    def _argmax_phase():
        base_g = i * _NG

        def body(j, carry):
            ma, ia, mb, ib = carry
            va = x0_ref[pl.ds(j * _G, _G), :]
            vb = x1_ref[pl.ds(j * _G, _G), :]
            jv = jnp.full((_G, _B), base_g + j, jnp.int32)
            ua = va > ma
            ub = vb > mb
            ma = jnp.where(ua, va, ma)
            ia = jnp.where(ua, jv, ia)
            mb = jnp.where(ub, vb, mb)
            ib = jnp.where(ub, jv, ib)
            return ma, ia, mb, ib

        ma, ia, mb, ib = lax.fori_loop(
            0, _NG, body,
            (m0[...], i0[...], m1[...], i1[...]), unroll=4,
        )
        m0[...] = ma
        i0[...] = ia
        m1[...] = mb
        i1[...] = ib

        @pl.when(i == _NB_IN - 1)
        def _collapse():
            r = jax.lax.broadcasted_iota(jnp.int32, (_G, _B), 0)
            res = jnp.zeros((1, _B), jnp.int32)
            for macc, vidx in ((m0[...], i0[...]), (m1[...], i1[...])):
                mx = jnp.max(macc, axis=0, keepdims=True)
                rows = vidx * _G + r
                cand = jnp.where(macc == mx, rows, jnp.int32(_BIG))
                res = res + jnp.min(cand, axis=0, keepdims=True)
            res_scr[...] = res

    @pl.when(i >= _NB_IN)
    def _onehot_phase():
        j = i - _NB_IN
        res = res_scr[...]
        row = jax.lax.broadcasted_iota(jnp.int32, (_RB_OUT, _B), 0) + j * _RB_OUT
        out_ref[...] = (row == res).astype(jnp.float32)


def kernel(x0, x1):
    out_t = pl.pallas_call(
        _fused_body,
        grid=(_NB_IN + _NB_OUT,),
        in_specs=[
            pl.BlockSpec((_RB_IN, _B), lambda i: (jnp.minimum(i, _NB_IN - 1), 0)),
            pl.BlockSpec((_RB_IN, _B), lambda i: (jnp.minimum(i, _NB_IN - 1), 0)),
        ],
        out_specs=pl.BlockSpec(
            (_RB_OUT, _B), lambda i: (jnp.maximum(i - _NB_IN, 0), 0)
        ),
        out_shape=jax.ShapeDtypeStruct((_OUT, _B), jnp.float32),
        scratch_shapes=[
            pltpu.VMEM((_G, _B), jnp.float32),
            pltpu.VMEM((_G, _B), jnp.int32),
            pltpu.VMEM((_G, _B), jnp.float32),
            pltpu.VMEM((_G, _B), jnp.int32),
            pltpu.VMEM((1, _B), jnp.int32),
        ],
    )(x0.T, x1.T)
    return out_t.T


# speedup vs baseline: 1.0154x; 1.0154x over previous
import jax
import jax.numpy as jnp
from jax import lax
from jax.experimental import pallas as pl
from jax.experimental.pallas import tpu as pltpu

_B = 128
_V = 100000
_OUT = 2 * _V - 1

_G = 16
_RB_IN = 10000
_NB_IN = _V // _RB_IN
_NG = _RB_IN // _G
_RB_OUT = 20000
_NB_OUT = (_OUT + _RB_OUT - 1) // _RB_OUT

_BIG = 2**30


def _fused_body(x0_ref, x1_ref, out_ref, m0, i0, m1, i1, res_scr):
    i = pl.program_id(0)

    @pl.when(i == 0)
    def _init():
        m0[...] = jnp.full_like(m0, -1.0)
        i0[...] = jnp.zeros_like(i0)
        m1[...] = jnp.full_like(m1, -1.0)
        i1[...] = jnp.zeros_like(i1)

    @pl.when(i < _NB_IN)
    def _argmax_phase():
        base_g = i * _NG

        def body(j, carry):
            ma, ia, mb, ib = carry
            va = x0_ref[pl.ds(j * _G, _G), :]
            vb = x1_ref[pl.ds(j * _G, _G), :]
            jv = jnp.full((_G, _B), base_g + j, jnp.int32)
            ua = va > ma
            ub = vb > mb
            ma = jnp.where(ua, va, ma)
            ia = jnp.where(ua, jv, ia)
            mb = jnp.where(ub, vb, mb)
            ib = jnp.where(ub, jv, ib)
            return ma, ia, mb, ib

        ma, ia, mb, ib = lax.fori_loop(
            0, _NG, body,
            (m0[...], i0[...], m1[...], i1[...]), unroll=4,
        )
        m0[...] = ma
        i0[...] = ia
        m1[...] = mb
        i1[...] = ib

        @pl.when(i == _NB_IN - 1)
        def _collapse():
            r = jax.lax.broadcasted_iota(jnp.int32, (_G, _B), 0)
            res = jnp.zeros((1, _B), jnp.int32)
            for macc, vidx in ((m0[...], i0[...]), (m1[...], i1[...])):
                mx = jnp.max(macc, axis=0, keepdims=True)
                rows = vidx * _G + r
                cand = jnp.where(macc == mx, rows, jnp.int32(_BIG))
                res = res + jnp.min(cand, axis=0, keepdims=True)
            res_scr[...] = res

    @pl.when(i >= _NB_IN)
    def _onehot_phase():
        j = i - _NB_IN
        res = res_scr[...]
        row = jax.lax.broadcasted_iota(jnp.int32, (_RB_OUT, _B), 0) + j * _RB_OUT
        out_ref[...] = (row == res).astype(jnp.float32)


def kernel(x0, x1):
    out_t = pl.pallas_call(
        _fused_body,
        grid=(_NB_IN + _NB_OUT,),
        in_specs=[
            pl.BlockSpec((_RB_IN, _B), lambda i: (jnp.minimum(i, _NB_IN - 1), 0)),
            pl.BlockSpec((_RB_IN, _B), lambda i: (jnp.minimum(i, _NB_IN - 1), 0)),
        ],
        out_specs=pl.BlockSpec(
            (_RB_OUT, _B), lambda i: (jnp.maximum(i - _NB_IN, 0), 0)
        ),
        out_shape=jax.ShapeDtypeStruct((_OUT, _B), jnp.float32),
        scratch_shapes=[
            pltpu.VMEM((_G, _B), jnp.float32),
            pltpu.VMEM((_G, _B), jnp.int32),
            pltpu.VMEM((_G, _B), jnp.float32),
            pltpu.VMEM((_G, _B), jnp.int32),
            pltpu.VMEM((1, _B), jnp.int32),
        ],
    )(x0.T, x1.T)
    return out_t.T
